# baseline (speedup 1.0000x reference)
import functools

import jax
import jax.numpy as jnp
from jax import lax
from jax.experimental import pallas as pl
from jax.experimental.pallas import tpu as pltpu
from jax.experimental.pallas import tpu_sc as plsc

S = 2048
H = 1024
E = 8
I = 2048

BLK = 256
NBLK = (2 * S + E * (BLK - 1) + BLK - 1) // BLK
CAP = NBLK * BLK

NC = 2
NS = 16
NW = NC * NS
TPW = S // NW


def _router_body(hs_ref, gw_ref, pos1_ref, pos2_ref, w1_ref, w2_ref, be_ref):
    x = hs_ref[...]
    gw = gw_ref[...]
    logits = lax.dot_general(x, gw, (((1,), (1,)), ((), ())),
                             preferred_element_type=jnp.float32)

    eio = lax.broadcasted_iota(jnp.int32, (S, E), 1)
    m1 = jnp.max(logits, axis=1, keepdims=True)
    i1 = jnp.min(jnp.where(logits == m1, eio, E), axis=1, keepdims=True)
    rest = jnp.where(eio == i1, -jnp.inf, logits)
    m2 = jnp.max(rest, axis=1, keepdims=True)
    i2 = jnp.min(jnp.where(rest == m2, eio, E), axis=1, keepdims=True)

    w1 = 1.0 / (1.0 + jnp.exp(m2 - m1))
    w1_ref[...] = w1
    w2_ref[...] = 1.0 - w1

    mask1 = (eio == i1).astype(jnp.float32)
    mask2 = (eio == i2).astype(jnp.float32)

    rio = lax.broadcasted_iota(jnp.int32, (S, S), 0)
    cio = lax.broadcasted_iota(jnp.int32, (S, S), 1)
    tri = (rio >= cio).astype(jnp.float32)
    cum1 = lax.dot_general(tri, mask1, (((1,), (0,)), ((), ())),
                           preferred_element_type=jnp.float32)
    cum2 = lax.dot_general(tri, mask2, (((1,), (0,)), ((), ())),
                           preferred_element_type=jnp.float32)

    c1 = jnp.sum(mask1, axis=0, keepdims=True)
    c2 = jnp.sum(mask2, axis=0, keepdims=True)
    loc1 = cum1 - 1.0
    loc2 = cum2 - 1.0 + c1
    total = (c1 + c2).astype(jnp.int32)
    padded = ((total + (BLK - 1)) // BLK) * BLK
    eri = lax.broadcasted_iota(jnp.int32, (E, E), 0)
    eci = lax.broadcasted_iota(jnp.int32, (E, E), 1)
    etri = (eri < eci).astype(jnp.float32)
    off = lax.dot_general(padded.astype(jnp.float32), etri,
                          (((1,), (0,)), ((), ())),
                          preferred_element_type=jnp.float32)

    pos1_ref[...] = jnp.sum(mask1 * (off + loc1), axis=1,
                            keepdims=True).astype(jnp.int32)
    pos2_ref[...] = jnp.sum(mask2 * (off + loc2), axis=1,
                            keepdims=True).astype(jnp.int32)

    ends = (off.astype(jnp.int32) + padded) // BLK
    bio = lax.broadcasted_iota(jnp.int32, (NBLK, E), 0)
    be = jnp.sum((jnp.broadcast_to(ends, (NBLK, E)) <= bio).astype(jnp.int32),
                 axis=1, keepdims=True)
    be_ref[...] = jnp.minimum(be, E - 1)


_router = pl.pallas_call(
    _router_body,
    out_shape=[
        jax.ShapeDtypeStruct((S, 1), jnp.int32),
        jax.ShapeDtypeStruct((S, 1), jnp.int32),
        jax.ShapeDtypeStruct((S, 1), jnp.float32),
        jax.ShapeDtypeStruct((S, 1), jnp.float32),
        jax.ShapeDtypeStruct((NBLK, 1), jnp.int32),
    ],
)


@functools.lru_cache(maxsize=None)
def _sc_kernels():
    mesh = plsc.VectorSubcoreMesh(core_axis_name="c", subcore_axis_name="s",
                                  num_cores=NC, num_subcores=NS)

    @functools.partial(
        pl.kernel,
        mesh=mesh,
        out_type=[
            jax.ShapeDtypeStruct((CAP, H), jnp.float32),
            jax.ShapeDtypeStruct((CAP,), jnp.float32),
        ],
        scratch_types=[
            pltpu.VMEM((TPW,), jnp.int32),
            pltpu.VMEM((TPW,), jnp.int32),
            pltpu.VMEM((TPW, H), jnp.float32),
            pltpu.VMEM((TPW,), jnp.float32),
            pltpu.VMEM((TPW,), jnp.float32),
            pltpu.SemaphoreType.DMA,
        ],
    )
    def dispatch(hs_hbm, pos1_hbm, pos2_hbm, w1_hbm, w2_hbm, disp_hbm,
                 wslot_hbm, idx1_v, idx2_v, rows_v, w1_v, w2_v, sem):
        wid = lax.axis_index("s") * NC + lax.axis_index("c")
        base = wid * TPW
        pltpu.sync_copy(hs_hbm.at[pl.ds(base, TPW)], rows_v)
        pltpu.sync_copy(pos1_hbm.at[pl.ds(base, TPW)], idx1_v)
        pltpu.sync_copy(pos2_hbm.at[pl.ds(base, TPW)], idx2_v)
        pltpu.sync_copy(w1_hbm.at[pl.ds(base, TPW)], w1_v)
        pltpu.sync_copy(w2_hbm.at[pl.ds(base, TPW)], w2_v)
        pltpu.async_copy(rows_v, disp_hbm.at[idx1_v], sem).wait()
        pltpu.async_copy(rows_v, disp_hbm.at[idx2_v], sem).wait()
        pltpu.async_copy(w1_v, wslot_hbm.at[idx1_v], sem).wait()
        pltpu.async_copy(w2_v, wslot_hbm.at[idx2_v], sem).wait()

    HALF = TPW // 2

    @functools.partial(
        pl.kernel,
        mesh=mesh,
        out_type=jax.ShapeDtypeStruct((S, H), jnp.float32),
        scratch_types=[
            pltpu.VMEM((HALF,), jnp.int32),
            pltpu.VMEM((HALF,), jnp.int32),
            pltpu.VMEM((HALF, H), jnp.float32),
            pltpu.VMEM((HALF, H), jnp.float32),
            pltpu.SemaphoreType.DMA,
            pltpu.SemaphoreType.DMA,
        ],
    )
    def combine(y_hbm, pos1_hbm, pos2_hbm, out_hbm, idx1_v, idx2_v, y1_v,
                y2_v, sem1, sem2):
        wid = lax.axis_index("s") * NC + lax.axis_index("c")
        base = wid * TPW
        for half in range(2):
            b = base + half * HALF
            pltpu.sync_copy(pos1_hbm.at[pl.ds(b, HALF)], idx1_v)
            pltpu.sync_copy(pos2_hbm.at[pl.ds(b, HALF)], idx2_v)
            cp1 = pltpu.async_copy(y_hbm.at[idx1_v], y1_v, sem1)
            cp2 = pltpu.async_copy(y_hbm.at[idx2_v], y2_v, sem2)
            cp1.wait()
            cp2.wait()

            def tok_body(t, _):
                def chunk_body(c, _):
                    sl = pl.ds(c * 16, 16)
                    y1_v[t, sl] = y1_v[t, sl] + y2_v[t, sl]
                    return 0

                return lax.fori_loop(0, H // 16, chunk_body, 0, unroll=4)

            lax.fori_loop(0, HALF, tok_body, 0)
            pltpu.sync_copy(y1_v, out_hbm.at[pl.ds(b, HALF)])

    return dispatch, combine


def _ffn_body(be_ref, x_ref, w13_ref, w2_ref, ws_ref, y_ref):
    x = x_ref[...].astype(jnp.bfloat16)
    w13 = w13_ref[0].astype(jnp.bfloat16)
    gu = lax.dot_general(x, w13, (((1,), (1,)), ((), ())),
                         preferred_element_type=jnp.float32)
    g = gu[:, :I]
    u = gu[:, I:]
    act = ((g * jax.nn.sigmoid(g)) * u).astype(jnp.bfloat16)
    w2e = w2_ref[0].astype(jnp.bfloat16)
    y = lax.dot_general(act, w2e, (((1,), (1,)), ((), ())),
                        preferred_element_type=jnp.float32)
    y_ref[...] = y * ws_ref[...]


_ffn = pl.pallas_call(
    _ffn_body,
    grid_spec=pltpu.PrefetchScalarGridSpec(
        num_scalar_prefetch=1,
        grid=(NBLK,),
        in_specs=[
            pl.BlockSpec((BLK, H), lambda i, be: (i, 0)),
            pl.BlockSpec((1, 2 * I, H), lambda i, be: (be[i], 0, 0)),
            pl.BlockSpec((1, H, I), lambda i, be: (be[i], 0, 0)),
            pl.BlockSpec((BLK, 1), lambda i, be: (i, 0)),
        ],
        out_specs=pl.BlockSpec((BLK, H), lambda i, be: (i, 0)),
    ),
    out_shape=jax.ShapeDtypeStruct((CAP, H), jnp.float32),
    compiler_params=pltpu.CompilerParams(
        dimension_semantics=("arbitrary",),
        vmem_limit_bytes=100 * 1024 * 1024,
    ),
)


def kernel(hidden_states, gate_weight, w13_weight, w2_weight):
    dispatch, combine = _sc_kernels()
    pos1, pos2, w1, w2, be = _router(hidden_states, gate_weight)
    pos1 = pos1.reshape(S)
    pos2 = pos2.reshape(S)
    disp, wslot = dispatch(hidden_states, pos1, pos2, w1.reshape(S),
                           w2.reshape(S))
    y = _ffn(be.reshape(NBLK), disp, w13_weight, w2_weight,
             wslot.reshape(CAP, 1))
    return combine(y, pos1, pos2)


# speedup vs baseline: 1.0627x; 1.0627x over previous
import functools

import jax
import jax.numpy as jnp
from jax import lax
from jax.experimental import pallas as pl
from jax.experimental.pallas import tpu as pltpu
from jax.experimental.pallas import tpu_sc as plsc

S = 2048
H = 1024
E = 8
I = 2048

BLK = 256
NBLK = (2 * S + E * (BLK - 1) + BLK - 1) // BLK
CAP = NBLK * BLK

NC = 2
NS = 16
NW = NC * NS
TPW = S // NW


def _router_body(hs_ref, gw_ref, pos1_ref, pos2_ref, w1_ref, w2_ref, be_ref,
                 first_ref, par_ref, next_ref, has_ref):
    x = hs_ref[...]
    gw = gw_ref[...]
    logits = lax.dot_general(x, gw, (((1,), (1,)), ((), ())),
                             preferred_element_type=jnp.float32)

    eio = lax.broadcasted_iota(jnp.int32, (S, E), 1)
    m1 = jnp.max(logits, axis=1, keepdims=True)
    i1 = jnp.min(jnp.where(logits == m1, eio, E), axis=1, keepdims=True)
    rest = jnp.where(eio == i1, -jnp.inf, logits)
    m2 = jnp.max(rest, axis=1, keepdims=True)
    i2 = jnp.min(jnp.where(rest == m2, eio, E), axis=1, keepdims=True)

    w1 = 1.0 / (1.0 + jnp.exp(m2 - m1))
    w1_ref[...] = w1
    w2_ref[...] = 1.0 - w1

    mask1 = (eio == i1).astype(jnp.float32)
    mask2 = (eio == i2).astype(jnp.float32)

    rio = lax.broadcasted_iota(jnp.int32, (S, S), 0)
    cio = lax.broadcasted_iota(jnp.int32, (S, S), 1)
    tri = (rio >= cio).astype(jnp.float32)
    cum1 = lax.dot_general(tri, mask1, (((1,), (0,)), ((), ())),
                           preferred_element_type=jnp.float32)
    cum2 = lax.dot_general(tri, mask2, (((1,), (0,)), ((), ())),
                           preferred_element_type=jnp.float32)

    c1 = jnp.sum(mask1, axis=0, keepdims=True)
    c2 = jnp.sum(mask2, axis=0, keepdims=True)
    loc1 = cum1 - 1.0
    loc2 = cum2 - 1.0 + c1
    total = (c1 + c2).astype(jnp.int32)
    padded = ((total + (BLK - 1)) // BLK) * BLK
    eri = lax.broadcasted_iota(jnp.int32, (E, E), 0)
    eci = lax.broadcasted_iota(jnp.int32, (E, E), 1)
    etri = (eri < eci).astype(jnp.float32)
    off = lax.dot_general(padded.astype(jnp.float32), etri,
                          (((1,), (0,)), ((), ())),
                          preferred_element_type=jnp.float32)

    pos1_ref[...] = jnp.sum(mask1 * (off + loc1), axis=1,
                            keepdims=True).astype(jnp.int32)
    pos2_ref[...] = jnp.sum(mask2 * (off + loc2), axis=1,
                            keepdims=True).astype(jnp.int32)

    ioff = off.astype(jnp.int32)
    ends = (ioff + padded) // BLK
    bio = lax.broadcasted_iota(jnp.int32, (NBLK, E), 0)
    eio_b = lax.broadcasted_iota(jnp.int32, (NBLK, E), 1)
    ends_b = jnp.broadcast_to(ends, (NBLK, E))
    be = jnp.sum((ends_b <= bio).astype(jnp.int32),
                 axis=1, keepdims=True)
    be = jnp.minimum(be, E - 1)
    be_ref[...] = be

    starts_b = jnp.broadcast_to(ioff // BLK, (NBLK, E))
    nonempty = jnp.broadcast_to(padded > 0, (NBLK, E))
    first = (jnp.sum(((starts_b == bio) & nonempty).astype(jnp.int32),
                     axis=1, keepdims=True) > 0).astype(jnp.int32)
    first_ref[...] = first
    bri = lax.broadcasted_iota(jnp.int32, (NBLK, NBLK), 0)
    bci = lax.broadcasted_iota(jnp.int32, (NBLK, NBLK), 1)
    btri = (bri >= bci).astype(jnp.float32)
    gidx = lax.dot_general(btri, first.astype(jnp.float32),
                           (((1,), (0,)), ((), ())),
                           preferred_element_type=jnp.float32)
    gidx = gidx.astype(jnp.int32) - 1
    par_ref[...] = lax.rem(gidx, 2)
    cand = (eio_b > jnp.broadcast_to(be, (NBLK, E))) & nonempty
    nxt = jnp.min(jnp.where(cand, eio_b, E), axis=1, keepdims=True)
    has_ref[...] = (nxt < E).astype(jnp.int32)
    next_ref[...] = jnp.minimum(nxt, E - 1)


_router = pl.pallas_call(
    _router_body,
    out_shape=[
        jax.ShapeDtypeStruct((S, 1), jnp.int32),
        jax.ShapeDtypeStruct((S, 1), jnp.int32),
        jax.ShapeDtypeStruct((S, 1), jnp.float32),
        jax.ShapeDtypeStruct((S, 1), jnp.float32),
        jax.ShapeDtypeStruct((NBLK, 1), jnp.int32),
        jax.ShapeDtypeStruct((NBLK, 1), jnp.int32),
        jax.ShapeDtypeStruct((NBLK, 1), jnp.int32),
        jax.ShapeDtypeStruct((NBLK, 1), jnp.int32),
        jax.ShapeDtypeStruct((NBLK, 1), jnp.int32),
    ],
)


@functools.lru_cache(maxsize=None)
def _sc_kernels():
    mesh = plsc.VectorSubcoreMesh(core_axis_name="c", subcore_axis_name="s",
                                  num_cores=NC, num_subcores=NS)

    @functools.partial(
        pl.kernel,
        mesh=mesh,
        out_type=[
            jax.ShapeDtypeStruct((CAP, H), jnp.float32),
            jax.ShapeDtypeStruct((CAP,), jnp.float32),
        ],
        scratch_types=[
            pltpu.VMEM((TPW,), jnp.int32),
            pltpu.VMEM((TPW,), jnp.int32),
            pltpu.VMEM((TPW, H), jnp.float32),
            pltpu.VMEM((TPW,), jnp.float32),
            pltpu.VMEM((TPW,), jnp.float32),
            pltpu.SemaphoreType.DMA,
        ],
    )
    def dispatch(hs_hbm, pos1_hbm, pos2_hbm, w1_hbm, w2_hbm, disp_hbm,
                 wslot_hbm, idx1_v, idx2_v, rows_v, w1_v, w2_v, sem):
        wid = lax.axis_index("s") * NC + lax.axis_index("c")
        base = wid * TPW
        pltpu.sync_copy(hs_hbm.at[pl.ds(base, TPW)], rows_v)
        pltpu.sync_copy(pos1_hbm.at[pl.ds(base, TPW)], idx1_v)
        pltpu.sync_copy(pos2_hbm.at[pl.ds(base, TPW)], idx2_v)
        pltpu.sync_copy(w1_hbm.at[pl.ds(base, TPW)], w1_v)
        pltpu.sync_copy(w2_hbm.at[pl.ds(base, TPW)], w2_v)
        pltpu.async_copy(rows_v, disp_hbm.at[idx1_v], sem).wait()
        pltpu.async_copy(rows_v, disp_hbm.at[idx2_v], sem).wait()
        pltpu.async_copy(w1_v, wslot_hbm.at[idx1_v], sem).wait()
        pltpu.async_copy(w2_v, wslot_hbm.at[idx2_v], sem).wait()

    CH = 16
    NCH = TPW // CH

    @functools.partial(
        pl.kernel,
        mesh=mesh,
        out_type=jax.ShapeDtypeStruct((S, H), jnp.float32),
        scratch_types=[
            pltpu.VMEM((TPW,), jnp.int32),
            pltpu.VMEM((TPW,), jnp.int32),
            pltpu.VMEM((2, CH, H), jnp.float32),
            pltpu.VMEM((2, CH, H), jnp.float32),
            pltpu.VMEM((CH, H), jnp.float32),
            pltpu.SemaphoreType.DMA((2,)),
            pltpu.SemaphoreType.DMA((2,)),
        ],
    )
    def combine(y_hbm, pos1_hbm, pos2_hbm, out_hbm, idx1_v, idx2_v, y1_v,
                y2_v, out_v, sem1, sem2):
        wid = lax.axis_index("s") * NC + lax.axis_index("c")
        base = wid * TPW
        pltpu.sync_copy(pos1_hbm.at[pl.ds(base, TPW)], idx1_v)
        pltpu.sync_copy(pos2_hbm.at[pl.ds(base, TPW)], idx2_v)

        def start(c):
            b = c % 2
            sl = pl.ds(c * CH, CH)
            pltpu.async_copy(y_hbm.at[idx1_v.at[sl]], y1_v.at[b], sem1.at[b])
            pltpu.async_copy(y_hbm.at[idx2_v.at[sl]], y2_v.at[b], sem2.at[b])

        def wait(c):
            b = c % 2
            pltpu.make_async_copy(y_hbm.at[idx1_v.at[pl.ds(c * CH, CH)]],
                                  y1_v.at[b], sem1.at[b]).wait()
            pltpu.make_async_copy(y_hbm.at[idx2_v.at[pl.ds(c * CH, CH)]],
                                  y2_v.at[b], sem2.at[b]).wait()

        start(0)
        for c in range(NCH):
            b = c % 2
            wait(c)
            if c + 1 < NCH:
                start(c + 1)

            def tok_body(t, _):
                def chunk_body(k, _):
                    sl = pl.ds(k * 16, 16)
                    out_v[t, sl] = y1_v[b, t, sl] + y2_v[b, t, sl]
                    return 0

                return lax.fori_loop(0, H // 16, chunk_body, 0, unroll=8)

            lax.fori_loop(0, CH, tok_body, 0)
            pltpu.sync_copy(out_v, out_hbm.at[pl.ds(base + c * CH, CH)])

    return dispatch, combine


def _ffn_body(be_ref, first_ref, par_ref, next_ref, has_ref,
              x_ref, w13_hbm, w2_hbm, ws_ref, y_ref,
              w13_v, w2_v, sem13, sem2):
    i = pl.program_id(0)
    first = first_ref[i]
    p = par_ref[i]

    @pl.when(i == 0)
    def _():
        e0 = be_ref[0]
        pltpu.make_async_copy(w13_hbm.at[e0], w13_v.at[0], sem13.at[0]).start()
        pltpu.make_async_copy(w2_hbm.at[e0], w2_v.at[0], sem2.at[0]).start()

    @pl.when((first == 1) & (has_ref[i] == 1))
    def _():
        ne = next_ref[i]
        q = 1 - p
        pltpu.make_async_copy(w13_hbm.at[ne], w13_v.at[q], sem13.at[q]).start()
        pltpu.make_async_copy(w2_hbm.at[ne], w2_v.at[q], sem2.at[q]).start()

    @pl.when(first == 1)
    def _():
        e = be_ref[i]
        pltpu.make_async_copy(w13_hbm.at[e], w13_v.at[p], sem13.at[p]).wait()
        pltpu.make_async_copy(w2_hbm.at[e], w2_v.at[p], sem2.at[p]).wait()

    x = x_ref[...].astype(jnp.bfloat16)
    w13 = w13_v[p].astype(jnp.bfloat16)
    gu = lax.dot_general(x, w13, (((1,), (1,)), ((), ())),
                         preferred_element_type=jnp.float32)
    g = gu[:, :I]
    u = gu[:, I:]
    act = ((g * jax.nn.sigmoid(g)) * u).astype(jnp.bfloat16)
    w2e = w2_v[p].astype(jnp.bfloat16)
    y = lax.dot_general(act, w2e, (((1,), (1,)), ((), ())),
                        preferred_element_type=jnp.float32)
    y_ref[...] = y * ws_ref[...]


_ffn = pl.pallas_call(
    _ffn_body,
    grid_spec=pltpu.PrefetchScalarGridSpec(
        num_scalar_prefetch=5,
        grid=(NBLK,),
        in_specs=[
            pl.BlockSpec((BLK, H), lambda i, *_: (i, 0)),
            pl.BlockSpec(memory_space=pl.ANY),
            pl.BlockSpec(memory_space=pl.ANY),
            pl.BlockSpec((BLK, 1), lambda i, *_: (i, 0)),
        ],
        out_specs=pl.BlockSpec((BLK, H), lambda i, *_: (i, 0)),
        scratch_shapes=[
            pltpu.VMEM((2, 2 * I, H), jnp.float32),
            pltpu.VMEM((2, H, I), jnp.float32),
            pltpu.SemaphoreType.DMA((2,)),
            pltpu.SemaphoreType.DMA((2,)),
        ],
    ),
    out_shape=jax.ShapeDtypeStruct((CAP, H), jnp.float32),
    compiler_params=pltpu.CompilerParams(
        dimension_semantics=("arbitrary",),
        vmem_limit_bytes=110 * 1024 * 1024,
    ),
)


def kernel(hidden_states, gate_weight, w13_weight, w2_weight):
    dispatch, combine = _sc_kernels()
    (pos1, pos2, w1, w2, be, first, par, nxt, has) = _router(
        hidden_states, gate_weight)
    pos1 = pos1.reshape(S)
    pos2 = pos2.reshape(S)
    disp, wslot = dispatch(hidden_states, pos1, pos2, w1.reshape(S),
                           w2.reshape(S))
    y = _ffn(be.reshape(NBLK), first.reshape(NBLK), par.reshape(NBLK),
             nxt.reshape(NBLK), has.reshape(NBLK),
             disp, w13_weight, w2_weight, wslot.reshape(CAP, 1))
    return combine(y, pos1, pos2)
